# baseline (speedup 1.0000x reference)
import functools

import jax
import jax.numpy as jnp
from jax import lax
from jax.experimental import pallas as pl
from jax.experimental.pallas import tpu as pltpu
from jax.experimental.pallas import tpu_sc as plsc

F32 = jnp.float32
I32 = jnp.int32

NC = 2
NS = 16
NW = NC * NS
EB = 128
DEG_WIN = 8


def _make_deg(NB, NPAD):
    TB = NB // NW
    rows_per = NPAD // NS
    mesh = plsc.VectorSubcoreMesh(core_axis_name="c", subcore_axis_name="s",
                                  num_cores=NC, num_subcores=NS)

    @functools.partial(
        pl.kernel,
        out_type=[jax.ShapeDtypeStruct((NPAD,), F32) for _ in range(NC)],
        mesh=mesh,
        scratch_types=[
            pltpu.VMEM((TB, EB), I32),
            pltpu.VMEM((EB,), F32),
            pltpu.VMEM((rows_per,), F32),
            pltpu.VMEM_SHARED((NPAD,), F32),
            pltpu.SemaphoreType.DMA,
        ],
    )
    def deg(dst_hbm, ones_hbm, zeros_hbm, out0, out1, idxb, onesb, zb, acc, sem):
        c = lax.axis_index("c")
        s = lax.axis_index("s")
        wid = c * NS + s

        pltpu.sync_copy(dst_hbm.at[pl.ds(wid * TB, TB)], idxb)
        pltpu.sync_copy(ones_hbm, onesb)
        pltpu.sync_copy(zeros_hbm, zb)
        pltpu.sync_copy(zb, acc.at[pl.ds(s * rows_per, rows_per)])
        plsc.subcore_barrier()

        def body(j, carry):
            pltpu.async_copy(onesb, acc.at[idxb.at[j]], sem, add=True)

            @pl.when(j >= DEG_WIN)
            def _():
                pltpu.make_async_copy(onesb, acc.at[idxb.at[0]], sem).wait()

            return carry

        lax.fori_loop(0, TB, body, 0)

        def drain(j, carry):
            pltpu.make_async_copy(onesb, acc.at[idxb.at[0]], sem).wait()
            return carry

        lax.fori_loop(0, min(DEG_WIN, TB), drain, 0)
        plsc.subcore_barrier()

        base = s * rows_per
        pltpu.sync_copy(acc.at[pl.ds(base, rows_per)], zb)

        @pl.when(c == 0)
        def _():
            pltpu.sync_copy(zb, out0.at[pl.ds(base, rows_per)])

        @pl.when(c == 1)
        def _():
            pltpu.sync_copy(zb, out1.at[pl.ds(base, rows_per)])

    return deg


IC = 16


def _make_agg(D, NB, NPAD):
    TB = NB // NW
    rows_per = NPAD // NS
    mesh = plsc.VectorSubcoreMesh(core_axis_name="c", subcore_axis_name="s",
                                  num_cores=NC, num_subcores=NS)

    @functools.partial(
        pl.kernel,
        out_type=[jax.ShapeDtypeStruct((NPAD, D), F32) for _ in range(NC)],
        mesh=mesh,
        scratch_types=[
            pltpu.VMEM((IC, EB), I32),
            pltpu.VMEM((IC, EB), I32),
            pltpu.VMEM((EB, D), F32),
            pltpu.VMEM((EB, D), F32),
            pltpu.VMEM_SHARED((NPAD, D), F32),
            pltpu.SemaphoreType.DMA,
            pltpu.SemaphoreType.DMA,
        ],
    )
    def agg(g_hbm, src_hbm, dst_hbm, zeros_hbm, out0, out1,
            srcb, dstb, rowsA, rowsB, acc, semA, semB):
        c = lax.axis_index("c")
        s = lax.axis_index("s")
        wid = c * NS + s

        pltpu.sync_copy(zeros_hbm, rowsA)
        for k in range(rows_per // EB):
            pltpu.sync_copy(rowsA, acc.at[pl.ds(s * rows_per + k * EB, EB)])
        plsc.subcore_barrier()

        def chunk(q, carry):
            base_blk = wid * TB + q * IC
            pltpu.sync_copy(src_hbm.at[pl.ds(base_blk, IC)], srcb)
            pltpu.sync_copy(dst_hbm.at[pl.ds(base_blk, IC)], dstb)
            pltpu.async_copy(g_hbm.at[srcb.at[0]], rowsA, semA)

            def body(i, c2):
                j = 2 * i
                pltpu.async_copy(g_hbm.at[srcb.at[j + 1]], rowsB, semB)
                pltpu.make_async_copy(g_hbm.at[srcb.at[0]], rowsA, semA).wait()
                pltpu.sync_copy(rowsA, acc.at[dstb.at[j]], add=True)

                @pl.when(i < IC // 2 - 1)
                def _():
                    pltpu.async_copy(g_hbm.at[srcb.at[j + 2]], rowsA, semA)

                pltpu.make_async_copy(g_hbm.at[srcb.at[0]], rowsB, semB).wait()
                pltpu.sync_copy(rowsB, acc.at[dstb.at[j + 1]], add=True)
                return c2

            lax.fori_loop(0, IC // 2, body, 0)
            return carry

        lax.fori_loop(0, TB // IC, chunk, 0)
        plsc.subcore_barrier()

        for k in range(rows_per // EB):
            base = s * rows_per + k * EB
            pltpu.sync_copy(acc.at[pl.ds(base, EB)], rowsA)

            @pl.when(c == 0)
            def _():
                pltpu.sync_copy(rowsA, out0.at[pl.ds(base, EB)])

            @pl.when(c == 1)
            def _():
                pltpu.sync_copy(rowsA, out1.at[pl.ds(base, EB)])

    return agg


def _tc1(dp0, dp1, x, W1, rb):
    N, DIN = x.shape
    DH = W1.shape[1]

    def body(d0, d1, xr, wr, dinv_ref, g_ref):
        deg = d0[...] + d1[...] + 1.0
        dv = lax.rsqrt(deg)
        dinv_ref[...] = dv
        g_ref[...] = jnp.dot(xr[...], wr[...], preferred_element_type=F32) * dv

    return pl.pallas_call(
        body,
        grid=(N // rb,),
        in_specs=[
            pl.BlockSpec((rb, 1), lambda i: (i, 0)),
            pl.BlockSpec((rb, 1), lambda i: (i, 0)),
            pl.BlockSpec((rb, DIN), lambda i: (i, 0)),
            pl.BlockSpec((DIN, DH), lambda i: (0, 0)),
        ],
        out_specs=[
            pl.BlockSpec((rb, 1), lambda i: (i, 0)),
            pl.BlockSpec((rb, DH), lambda i: (i, 0)),
        ],
        out_shape=[
            jax.ShapeDtypeStruct((N, 1), F32),
            jax.ShapeDtypeStruct((N, DH), F32),
        ],
    )(dp0, dp1, x, W1)


def _tc2(P0, P1, g1, dinv, b1, W2, rb):
    N, DH = g1.shape
    DO = W2.shape[1]

    def body(p0, p1, gr, dv, br, wr, out_ref):
        h = dv[...] * (p0[...] + p1[...] + gr[...]) + br[...]
        h = jnp.maximum(h, 0.0)
        out_ref[...] = jnp.dot(h, wr[...], preferred_element_type=F32) * dv[...]

    return pl.pallas_call(
        body,
        grid=(N // rb,),
        in_specs=[
            pl.BlockSpec((rb, DH), lambda i: (i, 0)),
            pl.BlockSpec((rb, DH), lambda i: (i, 0)),
            pl.BlockSpec((rb, DH), lambda i: (i, 0)),
            pl.BlockSpec((rb, 1), lambda i: (i, 0)),
            pl.BlockSpec((1, DH), lambda i: (0, 0)),
            pl.BlockSpec((DH, DO), lambda i: (0, 0)),
        ],
        out_specs=pl.BlockSpec((rb, DO), lambda i: (i, 0)),
        out_shape=jax.ShapeDtypeStruct((N, DO), F32),
    )(P0, P1, g1, dinv, b1, W2)


def _tc3(Q0, Q1, g2, dinv, b2, rb):
    N, DO = g2.shape

    def body(q0, q1, gr, dv, br, out_ref):
        z = dv[...] * (q0[...] + q1[...] + gr[...]) + br[...]
        m = jnp.max(z, axis=-1, keepdims=True)
        e = jnp.exp(z - m)
        lse = jnp.log(jnp.sum(e, axis=-1, keepdims=True))
        out_ref[...] = z - m - lse

    return pl.pallas_call(
        body,
        grid=(N // rb,),
        in_specs=[
            pl.BlockSpec((rb, DO), lambda i: (i, 0)),
            pl.BlockSpec((rb, DO), lambda i: (i, 0)),
            pl.BlockSpec((rb, DO), lambda i: (i, 0)),
            pl.BlockSpec((rb, 1), lambda i: (i, 0)),
            pl.BlockSpec((1, DO), lambda i: (0, 0)),
        ],
        out_specs=pl.BlockSpec((rb, DO), lambda i: (i, 0)),
        out_shape=jax.ShapeDtypeStruct((N, DO), F32),
    )(Q0, Q1, g2, dinv, b2)


def kernel(x, edge_index, W1, b1, W2, b2):
    N, DIN = x.shape
    DH = W1.shape[1]
    DO = W2.shape[1]
    E = edge_index.shape[1]

    NPAD = NS * EB * (-(-N // (NS * EB)))

    src = edge_index[0]
    dst = edge_index[1]
    CH = NW * EB * IC
    EPAD = CH * (-(-E // CH))
    if EPAD != E:
        pad = EPAD - E
        src = jnp.concatenate([src, jnp.zeros((pad,), I32)])
        dst = jnp.concatenate([dst, jnp.full((pad,), NPAD - 1, I32)])
    NB = EPAD // EB
    src2d = src.reshape(NB, EB)
    dst2d = dst.reshape(NB, EB)

    ones_eb = jnp.ones((EB,), F32)
    zeros_rp = jnp.zeros((NPAD // NS,), F32)

    dp0, dp1 = _make_deg(NB, NPAD)(dst2d, ones_eb, zeros_rp)
    dp0 = dp0[:N].reshape(N, 1)
    dp1 = dp1[:N].reshape(N, 1)

    rb = 1000
    dinv, g1 = _tc1(dp0, dp1, x, W1, rb)

    zeros_h = jnp.zeros((EB, DH), F32)
    P0, P1 = _make_agg(DH, NB, NPAD)(g1, src2d, dst2d, zeros_h)

    DOP = 128
    W2p = jnp.concatenate([W2, jnp.zeros((DH, DOP - DO), F32)], axis=1)
    g2p = _tc2(P0[:N], P1[:N], g1, dinv, b1.reshape(1, DH), W2p, rb)

    Q0, Q1 = _make_agg(DOP, NB, NPAD)(g2p, src2d, dst2d, zeros_h)
    return _tc3(Q0[:N, :DO], Q1[:N, :DO], g2p[:, :DO], dinv,
                b2.reshape(1, DO), rb)


# speedup vs baseline: 2.7953x; 2.7953x over previous
import functools

import jax
import jax.numpy as jnp
from jax import lax
from jax.experimental import pallas as pl
from jax.experimental.pallas import tpu as pltpu
from jax.experimental.pallas import tpu_sc as plsc

F32 = jnp.float32
I32 = jnp.int32

NC = 2
NS = 16
NW = NC * NS
EB = 128
DEG_WIN = 8


def _make_deg(NB, NPAD):
    TB = NB // NW
    rows_per = NPAD // NS
    mesh = plsc.VectorSubcoreMesh(core_axis_name="c", subcore_axis_name="s",
                                  num_cores=NC, num_subcores=NS)

    @functools.partial(
        pl.kernel,
        out_type=[jax.ShapeDtypeStruct((NPAD,), F32) for _ in range(NC)],
        mesh=mesh,
        scratch_types=[
            pltpu.VMEM((TB, EB), I32),
            pltpu.VMEM((EB,), F32),
            pltpu.VMEM((rows_per,), F32),
            pltpu.VMEM_SHARED((NPAD,), F32),
            pltpu.SemaphoreType.DMA,
        ],
    )
    def deg(dst_hbm, ones_hbm, zeros_hbm, out0, out1, idxb, onesb, zb, acc, sem):
        c = lax.axis_index("c")
        s = lax.axis_index("s")
        wid = c * NS + s

        pltpu.sync_copy(dst_hbm.at[pl.ds(wid * TB, TB)], idxb)
        pltpu.sync_copy(ones_hbm, onesb)
        pltpu.sync_copy(zeros_hbm, zb)
        pltpu.sync_copy(zb, acc.at[pl.ds(s * rows_per, rows_per)])
        plsc.subcore_barrier()

        def body(j, carry):
            pltpu.async_copy(onesb, acc.at[idxb.at[j]], sem, add=True)

            @pl.when(j >= DEG_WIN)
            def _():
                pltpu.make_async_copy(onesb, acc.at[idxb.at[0]], sem).wait()

            return carry

        lax.fori_loop(0, TB, body, 0)

        def drain(j, carry):
            pltpu.make_async_copy(onesb, acc.at[idxb.at[0]], sem).wait()
            return carry

        lax.fori_loop(0, min(DEG_WIN, TB), drain, 0)
        plsc.subcore_barrier()

        base = s * rows_per
        pltpu.sync_copy(acc.at[pl.ds(base, rows_per)], zb)

        @pl.when(c == 0)
        def _():
            pltpu.sync_copy(zb, out0.at[pl.ds(base, rows_per)])

        @pl.when(c == 1)
        def _():
            pltpu.sync_copy(zb, out1.at[pl.ds(base, rows_per)])

    return deg


IC = 16


def _make_agg(D, NB, NPAD):
    TB = NB // NW
    rows_per = NPAD // NS
    mesh = plsc.VectorSubcoreMesh(core_axis_name="c", subcore_axis_name="s",
                                  num_cores=NC, num_subcores=NS)

    @functools.partial(
        pl.kernel,
        out_type=[jax.ShapeDtypeStruct((NPAD, D), F32) for _ in range(NC)],
        mesh=mesh,
        scratch_types=[
            pltpu.VMEM((IC, EB), I32),
            pltpu.VMEM((IC, EB), I32),
            pltpu.VMEM((EB, D), F32),
            pltpu.VMEM((EB, D), F32),
            pltpu.VMEM_SHARED((NPAD, D), F32),
            pltpu.SemaphoreType.DMA,
            pltpu.SemaphoreType.DMA,
        ],
    )
    def agg(g_hbm, src_hbm, dst_hbm, zeros_hbm, out0, out1,
            srcb, dstb, rowsA, rowsB, acc, semA, semB):
        c = lax.axis_index("c")
        s = lax.axis_index("s")
        wid = c * NS + s

        pltpu.sync_copy(zeros_hbm, rowsA)
        for k in range(rows_per // EB):
            pltpu.sync_copy(rowsA, acc.at[pl.ds(s * rows_per + k * EB, EB)])
        plsc.subcore_barrier()

        def chunk(q, carry):
            base_blk = wid * TB + q * IC
            pltpu.sync_copy(src_hbm.at[pl.ds(base_blk, IC)], srcb)
            pltpu.sync_copy(dst_hbm.at[pl.ds(base_blk, IC)], dstb)
            pltpu.async_copy(g_hbm.at[srcb.at[0]], rowsA, semA)

            def body(i, c2):
                j = 2 * i
                pltpu.async_copy(g_hbm.at[srcb.at[j + 1]], rowsB, semB)
                pltpu.make_async_copy(g_hbm.at[srcb.at[0]], rowsA, semA).wait()
                pltpu.sync_copy(rowsA, acc.at[dstb.at[j]], add=True)

                @pl.when(i < IC // 2 - 1)
                def _():
                    pltpu.async_copy(g_hbm.at[srcb.at[j + 2]], rowsA, semA)

                pltpu.make_async_copy(g_hbm.at[srcb.at[0]], rowsB, semB).wait()
                pltpu.sync_copy(rowsB, acc.at[dstb.at[j + 1]], add=True)
                return c2

            lax.fori_loop(0, IC // 2, body, 0)
            return carry

        lax.fori_loop(0, TB // IC, chunk, 0)
        plsc.subcore_barrier()

        for k in range(rows_per // EB):
            base = s * rows_per + k * EB
            pltpu.sync_copy(acc.at[pl.ds(base, EB)], rowsA)

            @pl.when(c == 0)
            def _():
                pltpu.sync_copy(rowsA, out0.at[pl.ds(base, EB)])

            @pl.when(c == 1)
            def _():
                pltpu.sync_copy(rowsA, out1.at[pl.ds(base, EB)])

    return agg


def _tc1(dp0, dp1, x, W1, rb):
    N, DIN = x.shape
    DH = W1.shape[1]

    def body(d0, d1, xr, wr, dinv_ref, g_ref):
        deg = d0[...] + d1[...] + 1.0
        dv = lax.rsqrt(deg)
        dinv_ref[...] = dv
        g_ref[...] = jnp.dot(xr[...], wr[...], preferred_element_type=F32) * dv

    return pl.pallas_call(
        body,
        grid=(N // rb,),
        in_specs=[
            pl.BlockSpec((rb, 1), lambda i: (i, 0)),
            pl.BlockSpec((rb, 1), lambda i: (i, 0)),
            pl.BlockSpec((rb, DIN), lambda i: (i, 0)),
            pl.BlockSpec((DIN, DH), lambda i: (0, 0)),
        ],
        out_specs=[
            pl.BlockSpec((rb, 1), lambda i: (i, 0)),
            pl.BlockSpec((rb, DH), lambda i: (i, 0)),
        ],
        out_shape=[
            jax.ShapeDtypeStruct((N, 1), F32),
            jax.ShapeDtypeStruct((N, DH), F32),
        ],
    )(dp0, dp1, x, W1)


def _tc2(P0, P1, g1, dinv, b1, W2, rb):
    N, DH = g1.shape
    DO = W2.shape[1]

    def body(p0, p1, gr, dv, br, wr, out_ref):
        h = dv[...] * (p0[...] + p1[...] + gr[...]) + br[...]
        h = jnp.maximum(h, 0.0)
        out_ref[...] = jnp.dot(h, wr[...], preferred_element_type=F32) * dv[...]

    return pl.pallas_call(
        body,
        grid=(N // rb,),
        in_specs=[
            pl.BlockSpec((rb, DH), lambda i: (i, 0)),
            pl.BlockSpec((rb, DH), lambda i: (i, 0)),
            pl.BlockSpec((rb, DH), lambda i: (i, 0)),
            pl.BlockSpec((rb, 1), lambda i: (i, 0)),
            pl.BlockSpec((1, DH), lambda i: (0, 0)),
            pl.BlockSpec((DH, DO), lambda i: (0, 0)),
        ],
        out_specs=pl.BlockSpec((rb, DO), lambda i: (i, 0)),
        out_shape=jax.ShapeDtypeStruct((N, DO), F32),
    )(P0, P1, g1, dinv, b1, W2)


def _tc3(Q0, Q1, g2, dinv, b2, rb):
    N, DO = g2.shape

    def body(q0, q1, gr, dv, br, out_ref):
        z = dv[...] * (q0[...] + q1[...] + gr[...]) + br[...]
        m = jnp.max(z, axis=-1, keepdims=True)
        e = jnp.exp(z - m)
        lse = jnp.log(jnp.sum(e, axis=-1, keepdims=True))
        out_ref[...] = z - m - lse

    return pl.pallas_call(
        body,
        grid=(N // rb,),
        in_specs=[
            pl.BlockSpec((rb, DO), lambda i: (i, 0)),
            pl.BlockSpec((rb, DO), lambda i: (i, 0)),
            pl.BlockSpec((rb, DO), lambda i: (i, 0)),
            pl.BlockSpec((rb, 1), lambda i: (i, 0)),
            pl.BlockSpec((1, DO), lambda i: (0, 0)),
        ],
        out_specs=pl.BlockSpec((rb, DO), lambda i: (i, 0)),
        out_shape=jax.ShapeDtypeStruct((N, DO), F32),
    )(Q0, Q1, g2, dinv, b2)


def kernel(x, edge_index, W1, b1, W2, b2):
    N, DIN = x.shape
    DH = W1.shape[1]
    DO = W2.shape[1]
    E = edge_index.shape[1]

    NPAD = NS * EB * (-(-N // (NS * EB)))

    src = edge_index[0]
    dst = edge_index[1]
    CH = NW * EB * IC
    EPAD = CH * (-(-E // CH))
    if EPAD != E:
        pad = EPAD - E
        pidx = jnp.arange(pad, dtype=I32)
        src = jnp.concatenate([src, pidx % N])
        dst = jnp.concatenate([dst, N + pidx % (NPAD - N)])
    NB = EPAD // EB
    src2d = src.reshape(NB, EB)
    dst2d = dst.reshape(NB, EB)

    ones_eb = jnp.ones((EB,), F32)
    zeros_rp = jnp.zeros((NPAD // NS,), F32)

    dp0, dp1 = _make_deg(NB, NPAD)(dst2d, ones_eb, zeros_rp)
    dp0 = dp0[:N].reshape(N, 1)
    dp1 = dp1[:N].reshape(N, 1)

    rb = 1000
    dinv, g1 = _tc1(dp0, dp1, x, W1, rb)

    zeros_h = jnp.zeros((EB, DH), F32)
    P0, P1 = _make_agg(DH, NB, NPAD)(g1, src2d, dst2d, zeros_h)

    DOP = 128
    W2p = jnp.concatenate([W2, jnp.zeros((DH, DOP - DO), F32)], axis=1)
    g2p = _tc2(P0[:N], P1[:N], g1, dinv, b1.reshape(1, DH), W2p, rb)

    Q0, Q1 = _make_agg(DOP, NB, NPAD)(g2p, src2d, dst2d, zeros_h)
    return _tc3(Q0[:N, :DO], Q1[:N, :DO], g2p[:, :DO], dinv,
                b2.reshape(1, DO), rb)


# speedup vs baseline: 2.9387x; 1.0513x over previous
import functools

import jax
import jax.numpy as jnp
from jax import lax
from jax.experimental import pallas as pl
from jax.experimental.pallas import tpu as pltpu
from jax.experimental.pallas import tpu_sc as plsc

F32 = jnp.float32
I32 = jnp.int32

NC = 2
NS = 16
NW = NC * NS
EB = 128
DEG_WIN = 8


def _make_deg(NB, NPAD):
    TB = NB // NW
    rows_per = NPAD // NS
    mesh = plsc.VectorSubcoreMesh(core_axis_name="c", subcore_axis_name="s",
                                  num_cores=NC, num_subcores=NS)

    @functools.partial(
        pl.kernel,
        out_type=[jax.ShapeDtypeStruct((NPAD,), F32) for _ in range(NC)],
        mesh=mesh,
        scratch_types=[
            pltpu.VMEM((TB, EB), I32),
            pltpu.VMEM((EB,), F32),
            pltpu.VMEM((rows_per,), F32),
            pltpu.VMEM_SHARED((NPAD,), F32),
            pltpu.SemaphoreType.DMA,
        ],
    )
    def deg(dst_hbm, ones_hbm, zeros_hbm, out0, out1, idxb, onesb, zb, acc, sem):
        c = lax.axis_index("c")
        s = lax.axis_index("s")
        wid = c * NS + s

        pltpu.sync_copy(dst_hbm.at[pl.ds(wid * TB, TB)], idxb)
        pltpu.sync_copy(ones_hbm, onesb)
        pltpu.sync_copy(zeros_hbm, zb)
        pltpu.sync_copy(zb, acc.at[pl.ds(s * rows_per, rows_per)])
        plsc.subcore_barrier()

        def body(j, carry):
            pltpu.async_copy(onesb, acc.at[idxb.at[j]], sem, add=True)

            @pl.when(j >= DEG_WIN)
            def _():
                pltpu.make_async_copy(onesb, acc.at[idxb.at[0]], sem).wait()

            return carry

        lax.fori_loop(0, TB, body, 0)

        def drain(j, carry):
            pltpu.make_async_copy(onesb, acc.at[idxb.at[0]], sem).wait()
            return carry

        lax.fori_loop(0, min(DEG_WIN, TB), drain, 0)
        plsc.subcore_barrier()

        base = s * rows_per
        pltpu.sync_copy(acc.at[pl.ds(base, rows_per)], zb)

        @pl.when(c == 0)
        def _():
            pltpu.sync_copy(zb, out0.at[pl.ds(base, rows_per)])

        @pl.when(c == 1)
        def _():
            pltpu.sync_copy(zb, out1.at[pl.ds(base, rows_per)])

    return deg


IC = 16


def _make_agg(D, NB, NPAD):
    TB = NB // NW
    rows_per = NPAD // NS
    mesh = plsc.VectorSubcoreMesh(core_axis_name="c", subcore_axis_name="s",
                                  num_cores=NC, num_subcores=NS)

    @functools.partial(
        pl.kernel,
        out_type=[jax.ShapeDtypeStruct((NPAD, D), F32) for _ in range(NC)],
        mesh=mesh,
        scratch_types=[
            pltpu.VMEM((IC, EB), I32),
            pltpu.VMEM((IC, EB), I32),
            pltpu.VMEM((EB, D), F32),
            pltpu.VMEM((EB, D), F32),
            pltpu.VMEM_SHARED((NPAD, D), F32),
            pltpu.SemaphoreType.DMA,
            pltpu.SemaphoreType.DMA,
        ],
    )
    def agg(g_hbm, src_hbm, dst_hbm, zeros_hbm, out0, out1,
            srcb, dstb, rowsA, rowsB, acc, semA, semB):
        c = lax.axis_index("c")
        s = lax.axis_index("s")
        wid = c * NS + s

        pltpu.sync_copy(zeros_hbm, rowsA)
        for k in range(rows_per // EB):
            pltpu.sync_copy(rowsA, acc.at[pl.ds(s * rows_per + k * EB, EB)])
        plsc.subcore_barrier()

        def chunk(q, carry):
            base_blk = wid * TB + q * IC
            pltpu.sync_copy(src_hbm.at[pl.ds(base_blk, IC)], srcb)
            pltpu.sync_copy(dst_hbm.at[pl.ds(base_blk, IC)], dstb)
            pltpu.async_copy(g_hbm.at[srcb.at[0]], rowsA, semA)

            def body(i, c2):
                j = 2 * i
                pltpu.async_copy(g_hbm.at[srcb.at[j + 1]], rowsB, semB)
                pltpu.make_async_copy(g_hbm.at[srcb.at[0]], rowsA, semA).wait()
                pltpu.sync_copy(rowsA, acc.at[dstb.at[j]], add=True)

                @pl.when(i < IC // 2 - 1)
                def _():
                    pltpu.async_copy(g_hbm.at[srcb.at[j + 2]], rowsA, semA)

                pltpu.make_async_copy(g_hbm.at[srcb.at[0]], rowsB, semB).wait()
                pltpu.sync_copy(rowsB, acc.at[dstb.at[j + 1]], add=True)
                return c2

            lax.fori_loop(0, IC // 2, body, 0)
            return carry

        lax.fori_loop(0, TB // IC, chunk, 0)
        plsc.subcore_barrier()

        for k in range(rows_per // EB):
            base = s * rows_per + k * EB
            pltpu.sync_copy(acc.at[pl.ds(base, EB)], rowsA)

            @pl.when(c == 0)
            def _():
                pltpu.sync_copy(rowsA, out0.at[pl.ds(base, EB)])

            @pl.when(c == 1)
            def _():
                pltpu.sync_copy(rowsA, out1.at[pl.ds(base, EB)])

    return agg


def _tc1(dp0, dp1, x, W1, rb):
    N, DIN = x.shape
    DH = W1.shape[1]

    def body(d0, d1, xr, wr, dinv_ref, g_ref):
        deg = d0[...] + d1[...] + 1.0
        dv = lax.rsqrt(deg)
        dinv_ref[...] = dv
        g_ref[...] = jnp.dot(xr[...], wr[...], preferred_element_type=F32) * dv

    return pl.pallas_call(
        body,
        grid=(N // rb,),
        in_specs=[
            pl.BlockSpec((rb, 1), lambda i: (i, 0)),
            pl.BlockSpec((rb, 1), lambda i: (i, 0)),
            pl.BlockSpec((rb, DIN), lambda i: (i, 0)),
            pl.BlockSpec((DIN, DH), lambda i: (0, 0)),
        ],
        out_specs=[
            pl.BlockSpec((rb, 1), lambda i: (i, 0)),
            pl.BlockSpec((rb, DH), lambda i: (i, 0)),
        ],
        out_shape=[
            jax.ShapeDtypeStruct((N, 1), F32),
            jax.ShapeDtypeStruct((N, DH), F32),
        ],
    )(dp0, dp1, x, W1)


def _tc2(P0, P1, g1, dinv, b1, W2, rb):
    N, DH = g1.shape
    DO = W2.shape[1]

    def body(p0, p1, gr, dv, br, wr, out_ref):
        h = dv[...] * (p0[...] + p1[...] + gr[...]) + br[...]
        h = jnp.maximum(h, 0.0)
        out_ref[...] = jnp.dot(h, wr[...], preferred_element_type=F32) * dv[...]

    return pl.pallas_call(
        body,
        grid=(N // rb,),
        in_specs=[
            pl.BlockSpec((rb, DH), lambda i: (i, 0)),
            pl.BlockSpec((rb, DH), lambda i: (i, 0)),
            pl.BlockSpec((rb, DH), lambda i: (i, 0)),
            pl.BlockSpec((rb, 1), lambda i: (i, 0)),
            pl.BlockSpec((1, DH), lambda i: (0, 0)),
            pl.BlockSpec((DH, DO), lambda i: (0, 0)),
        ],
        out_specs=pl.BlockSpec((rb, DO), lambda i: (i, 0)),
        out_shape=jax.ShapeDtypeStruct((N, DO), F32),
    )(P0, P1, g1, dinv, b1, W2)


def _tc3(Q0, Q1, g2, dinv, b2, rb):
    N = g2.shape[0]
    DP = g2.shape[1]
    DO = b2.shape[1]

    def body(q0, q1, gr, dv, br, out_ref):
        z = (dv[...] * (q0[...] + q1[...] + gr[...]))[:, :DO] + br[...]
        m = jnp.max(z, axis=-1, keepdims=True)
        e = jnp.exp(z - m)
        lse = jnp.log(jnp.sum(e, axis=-1, keepdims=True))
        out_ref[...] = z - m - lse

    return pl.pallas_call(
        body,
        grid=(N // rb,),
        in_specs=[
            pl.BlockSpec((rb, DP), lambda i: (i, 0)),
            pl.BlockSpec((rb, DP), lambda i: (i, 0)),
            pl.BlockSpec((rb, DP), lambda i: (i, 0)),
            pl.BlockSpec((rb, 1), lambda i: (i, 0)),
            pl.BlockSpec((1, DO), lambda i: (0, 0)),
        ],
        out_specs=pl.BlockSpec((rb, DO), lambda i: (i, 0)),
        out_shape=jax.ShapeDtypeStruct((N, DO), F32),
    )(Q0, Q1, g2, dinv, b2)


def kernel(x, edge_index, W1, b1, W2, b2):
    N, DIN = x.shape
    DH = W1.shape[1]
    DO = W2.shape[1]
    E = edge_index.shape[1]

    NPAD = NS * EB * (-(-N // (NS * EB)))

    src = edge_index[0]
    dst = edge_index[1]
    CH = NW * EB * IC
    EPAD = CH * (-(-E // CH))
    if EPAD != E:
        pad = EPAD - E
        pidx = jnp.arange(pad, dtype=I32)
        src = jnp.concatenate([src, pidx % N])
        dst = jnp.concatenate([dst, N + pidx % (NPAD - N)])
    NB = EPAD // EB
    src2d = src.reshape(NB, EB)
    dst2d = dst.reshape(NB, EB)

    ones_eb = jnp.ones((EB,), F32)
    zeros_rp = jnp.zeros((NPAD // NS,), F32)

    dp0, dp1 = _make_deg(NB, NPAD)(dst2d, ones_eb, zeros_rp)
    dp0 = dp0[:N].reshape(N, 1)
    dp1 = dp1[:N].reshape(N, 1)

    rb = 1000
    dinv, g1 = _tc1(dp0, dp1, x, W1, rb)

    zeros_h = jnp.zeros((EB, DH), F32)
    P0, P1 = _make_agg(DH, NB, NPAD)(g1, src2d, dst2d, zeros_h)

    DOP = 128
    W2p = jnp.concatenate([W2, jnp.zeros((DH, DOP - DO), F32)], axis=1)
    g2p = _tc2(P0, P1, g1, dinv, b1.reshape(1, DH), W2p, rb)

    Q0, Q1 = _make_agg(DOP, NB, NPAD)(g2p, src2d, dst2d, zeros_h)
    return _tc3(Q0, Q1, g2p, dinv, b2.reshape(1, DO), rb)


# speedup vs baseline: 3.2357x; 1.1011x over previous
import functools

import jax
import jax.numpy as jnp
from jax import lax
from jax.experimental import pallas as pl
from jax.experimental.pallas import tpu as pltpu
from jax.experimental.pallas import tpu_sc as plsc

F32 = jnp.float32
I32 = jnp.int32

NC = 2
NS = 16
NW = NC * NS
EB = 128
DEG_WIN = 8


def _make_deg(NB, NPAD):
    TB = NB // NW
    rows_per = NPAD // NS
    mesh = plsc.VectorSubcoreMesh(core_axis_name="c", subcore_axis_name="s",
                                  num_cores=NC, num_subcores=NS)

    @functools.partial(
        pl.kernel,
        out_type=[jax.ShapeDtypeStruct((NPAD,), F32) for _ in range(NC)],
        mesh=mesh,
        scratch_types=[
            pltpu.VMEM((TB, EB), I32),
            pltpu.VMEM((EB,), F32),
            pltpu.VMEM((rows_per,), F32),
            pltpu.VMEM_SHARED((NPAD,), F32),
            pltpu.SemaphoreType.DMA,
        ],
    )
    def deg(dst_hbm, ones_hbm, zeros_hbm, out0, out1, idxb, onesb, zb, acc, sem):
        c = lax.axis_index("c")
        s = lax.axis_index("s")
        wid = c * NS + s

        pltpu.sync_copy(dst_hbm.at[pl.ds(wid * TB, TB)], idxb)
        pltpu.sync_copy(ones_hbm, onesb)
        pltpu.sync_copy(zeros_hbm, zb)
        pltpu.sync_copy(zb, acc.at[pl.ds(s * rows_per, rows_per)])
        plsc.subcore_barrier()

        def body(j, carry):
            pltpu.async_copy(onesb, acc.at[idxb.at[j]], sem, add=True)

            @pl.when(j >= DEG_WIN)
            def _():
                pltpu.make_async_copy(onesb, acc.at[idxb.at[0]], sem).wait()

            return carry

        lax.fori_loop(0, TB, body, 0)

        def drain(j, carry):
            pltpu.make_async_copy(onesb, acc.at[idxb.at[0]], sem).wait()
            return carry

        lax.fori_loop(0, min(DEG_WIN, TB), drain, 0)
        plsc.subcore_barrier()

        base = s * rows_per
        pltpu.sync_copy(acc.at[pl.ds(base, rows_per)], zb)

        @pl.when(c == 0)
        def _():
            pltpu.sync_copy(zb, out0.at[pl.ds(base, rows_per)])

        @pl.when(c == 1)
        def _():
            pltpu.sync_copy(zb, out1.at[pl.ds(base, rows_per)])

    return deg


IC = 16


def _make_agg(D, NB, NPAD, tc_tiling=True):
    TB = NB // NW
    rows_per = NPAD // NS
    mesh = plsc.VectorSubcoreMesh(core_axis_name="c", subcore_axis_name="s",
                                  num_cores=NC, num_subcores=NS)

    @functools.partial(
        pl.kernel,
        out_type=[jax.ShapeDtypeStruct((NPAD, D), F32) for _ in range(NC)],
        mesh=mesh,
        compiler_params=pltpu.CompilerParams(use_tc_tiling_on_sc=tc_tiling),
        scratch_types=[
            pltpu.VMEM((IC, EB), I32),
            pltpu.VMEM((IC, EB), I32),
            pltpu.VMEM((EB, D), F32),
            pltpu.VMEM((EB, D), F32),
            pltpu.VMEM_SHARED((NPAD, D), F32),
            pltpu.SemaphoreType.DMA,
            pltpu.SemaphoreType.DMA,
        ],
    )
    def agg(g_hbm, src_hbm, dst_hbm, zeros_hbm, out0, out1,
            srcb, dstb, rowsA, rowsB, acc, semA, semB):
        c = lax.axis_index("c")
        s = lax.axis_index("s")
        wid = c * NS + s

        pltpu.sync_copy(zeros_hbm, rowsA)
        for k in range(rows_per // EB):
            pltpu.sync_copy(rowsA, acc.at[pl.ds(s * rows_per + k * EB, EB)])
        plsc.subcore_barrier()

        def chunk(q, carry):
            base_blk = wid * TB + q * IC
            pltpu.sync_copy(src_hbm.at[pl.ds(base_blk, IC)], srcb)
            pltpu.sync_copy(dst_hbm.at[pl.ds(base_blk, IC)], dstb)
            pltpu.async_copy(g_hbm.at[srcb.at[0]], rowsA, semA)

            def body(i, c2):
                j = 2 * i
                pltpu.async_copy(g_hbm.at[srcb.at[j + 1]], rowsB, semB)
                pltpu.make_async_copy(g_hbm.at[srcb.at[0]], rowsA, semA).wait()
                pltpu.sync_copy(rowsA, acc.at[dstb.at[j]], add=True)

                @pl.when(i < IC // 2 - 1)
                def _():
                    pltpu.async_copy(g_hbm.at[srcb.at[j + 2]], rowsA, semA)

                pltpu.make_async_copy(g_hbm.at[srcb.at[0]], rowsB, semB).wait()
                pltpu.sync_copy(rowsB, acc.at[dstb.at[j + 1]], add=True)
                return c2

            lax.fori_loop(0, IC // 2, body, 0)
            return carry

        lax.fori_loop(0, TB // IC, chunk, 0)
        plsc.subcore_barrier()

        for k in range(rows_per // EB):
            base = s * rows_per + k * EB
            pltpu.sync_copy(acc.at[pl.ds(base, EB)], rowsA)

            @pl.when(c == 0)
            def _():
                pltpu.sync_copy(rowsA, out0.at[pl.ds(base, EB)])

            @pl.when(c == 1)
            def _():
                pltpu.sync_copy(rowsA, out1.at[pl.ds(base, EB)])

    return agg


def _tc1(dp0, dp1, x, W1, rb):
    N, DIN = x.shape
    DH = W1.shape[1]

    def body(d0, d1, xr, wr, dinv_ref, g_ref):
        deg = d0[...] + d1[...] + 1.0
        dv = lax.rsqrt(deg)
        dinv_ref[...] = dv
        g_ref[...] = jnp.dot(xr[...], wr[...], preferred_element_type=F32) * dv

    return pl.pallas_call(
        body,
        grid=(N // rb,),
        in_specs=[
            pl.BlockSpec((rb, 1), lambda i: (i, 0)),
            pl.BlockSpec((rb, 1), lambda i: (i, 0)),
            pl.BlockSpec((rb, DIN), lambda i: (i, 0)),
            pl.BlockSpec((DIN, DH), lambda i: (0, 0)),
        ],
        out_specs=[
            pl.BlockSpec((rb, 1), lambda i: (i, 0)),
            pl.BlockSpec((rb, DH), lambda i: (i, 0)),
        ],
        out_shape=[
            jax.ShapeDtypeStruct((N, 1), F32),
            jax.ShapeDtypeStruct((N, DH), F32),
        ],
    )(dp0, dp1, x, W1)


def _tc2(P0, P1, g1, dinv, b1, W2, rb):
    N, DH = g1.shape
    DO = W2.shape[1]

    def body(p0, p1, gr, dv, br, wr, out_ref):
        h = dv[...] * (p0[...] + p1[...] + gr[...]) + br[...]
        h = jnp.maximum(h, 0.0)
        out_ref[...] = jnp.dot(h, wr[...], preferred_element_type=F32) * dv[...]

    return pl.pallas_call(
        body,
        grid=(N // rb,),
        in_specs=[
            pl.BlockSpec((rb, DH), lambda i: (i, 0)),
            pl.BlockSpec((rb, DH), lambda i: (i, 0)),
            pl.BlockSpec((rb, DH), lambda i: (i, 0)),
            pl.BlockSpec((rb, 1), lambda i: (i, 0)),
            pl.BlockSpec((1, DH), lambda i: (0, 0)),
            pl.BlockSpec((DH, DO), lambda i: (0, 0)),
        ],
        out_specs=pl.BlockSpec((rb, DO), lambda i: (i, 0)),
        out_shape=jax.ShapeDtypeStruct((N, DO), F32),
    )(P0, P1, g1, dinv, b1, W2)


def _tc3(Q0, Q1, g2, dinv, b2, rb):
    N = g2.shape[0]
    DP = g2.shape[1]
    DO = b2.shape[1]

    def body(q0, q1, gr, dv, br, out_ref):
        z = (dv[...] * (q0[...] + q1[...] + gr[...]))[:, :DO] + br[...]
        m = jnp.max(z, axis=-1, keepdims=True)
        e = jnp.exp(z - m)
        lse = jnp.log(jnp.sum(e, axis=-1, keepdims=True))
        out_ref[...] = z - m - lse

    return pl.pallas_call(
        body,
        grid=(N // rb,),
        in_specs=[
            pl.BlockSpec((rb, DP), lambda i: (i, 0)),
            pl.BlockSpec((rb, DP), lambda i: (i, 0)),
            pl.BlockSpec((rb, DP), lambda i: (i, 0)),
            pl.BlockSpec((rb, 1), lambda i: (i, 0)),
            pl.BlockSpec((1, DO), lambda i: (0, 0)),
        ],
        out_specs=pl.BlockSpec((rb, DO), lambda i: (i, 0)),
        out_shape=jax.ShapeDtypeStruct((N, DO), F32),
    )(Q0, Q1, g2, dinv, b2)


def kernel(x, edge_index, W1, b1, W2, b2):
    N, DIN = x.shape
    DH = W1.shape[1]
    DO = W2.shape[1]
    E = edge_index.shape[1]

    NPAD = NS * EB * (-(-N // (NS * EB)))

    src = edge_index[0]
    dst = edge_index[1]
    CH = NW * EB * IC
    EPAD = CH * (-(-E // CH))
    if EPAD != E:
        pad = EPAD - E
        pidx = jnp.arange(pad, dtype=I32)
        src = jnp.concatenate([src, pidx % N])
        dst = jnp.concatenate([dst, N + pidx % (NPAD - N)])
    NB = EPAD // EB
    src2d = src.reshape(NB, EB)
    dst2d = dst.reshape(NB, EB)

    ones_eb = jnp.ones((EB,), F32)
    zeros_rp = jnp.zeros((NPAD // NS,), F32)

    dp0, dp1 = _make_deg(NB, NPAD)(dst2d, ones_eb, zeros_rp)
    dp0 = dp0[:N].reshape(N, 1)
    dp1 = dp1[:N].reshape(N, 1)

    rb = 1000
    dinv, g1 = _tc1(dp0, dp1, x, W1, rb)

    zeros_h = jnp.zeros((EB, DH), F32)
    P0, P1 = _make_agg(DH, NB, NPAD)(g1, src2d, dst2d, zeros_h)

    g2 = _tc2(P0, P1, g1, dinv, b1.reshape(1, DH), W2, rb)

    zeros_o = jnp.zeros((EB, DO), F32)
    Q0, Q1 = _make_agg(DO, NB, NPAD, tc_tiling=False)(g2, src2d, dst2d, zeros_o)
    return _tc3(Q0, Q1, g2, dinv, b2.reshape(1, DO), rb)
